# baseline (speedup 1.0000x reference)
import functools

import jax
import jax.numpy as jnp
from jax import lax
from jax.experimental import pallas as pl
from jax.experimental.pallas import tpu as pltpu
from jax.experimental.pallas import tpu_sc as plsc

N_TRAIN = 20000
W = 256
N_TEST = 1024
BC = 2048
N_PAD = 20480
N_TILES = N_PAD // BC
BT = 512
LANES = 128
CHUNKS = BC // LANES
N_BLOCKS = N_PAD // LANES
CLASS_SIZE = 2000
MAX_DISTANCE = 256
SCALE = 16384.0
BIG = 3.0e7

N_WORKERS = 32
RPW = N_TEST // N_WORKERS


def _tc_body(test_ref, train_ref, colv_ref, keys_ref, sel_ref, bscr):
    t = pl.program_id(0)
    i = pl.program_id(1)
    dot = jax.lax.dot_general(
        test_ref[...], train_ref[...],
        (((1,), (1,)), ((), ())),
        preferred_element_type=jnp.float32,
    )
    keys = dot + colv_ref[...]

    rows = pl.ds(i * BT, BT)

    @pl.when(t == 0)
    def _():
        bscr[rows, :] = jnp.full((BT, N_TILES * LANES), BIG, jnp.float32)

    bms = []
    for c in range(CHUNKS):
        x = keys[:, c * LANES:(c + 1) * LANES]
        keys_ref[c, :, :] = x
        bms.append(jnp.min(x, axis=1, keepdims=True))
    lane0 = pl.multiple_of(t * LANES, LANES)
    bscr[rows, pl.ds(lane0, CHUNKS)] = jnp.concatenate(bms, axis=1)

    @pl.when(t == N_TILES - 1)
    def _():
        bmin = bscr[rows, :]
        m1 = jnp.min(bmin, axis=1, keepdims=True)
        b2 = jnp.where(bmin == m1, BIG, bmin)
        m2 = jnp.min(b2, axis=1, keepdims=True)
        b3 = jnp.where(b2 == m2, BIG, b2)
        m3 = jnp.min(b3, axis=1, keepdims=True)

        row = (i * BT + lax.broadcasted_iota(jnp.int32, (BT, 1), 0))

        def blockrow(key_f):
            ki = key_f.astype(jnp.int32)
            block = (ki & 32767) >> 7
            return block * N_TEST + row

        id1, id2, id3 = blockrow(m1), blockrow(m2), blockrow(m3)
        sel_ref[...] = jnp.concatenate([id1, id2, id3] + [id1] * 13, axis=1)


@jax.jit
def _tc_keys(test_in, train_in, colvec):
    keys3d, sel = pl.pallas_call(
        _tc_body,
        grid=(N_TILES, N_TEST // BT),
        in_specs=[
            pl.BlockSpec((BT, W), lambda t, i: (i, 0)),
            pl.BlockSpec((BC, W), lambda t, i: (t, 0)),
            pl.BlockSpec((1, BC), lambda t, i: (0, t)),
        ],
        out_specs=[
            pl.BlockSpec((CHUNKS, BT, LANES), lambda t, i: (t, i, 0)),
            pl.BlockSpec((BT, 16), lambda t, i: (i, 0)),
        ],
        out_shape=[
            jax.ShapeDtypeStruct((N_BLOCKS, N_TEST, LANES), jnp.float32),
            jax.ShapeDtypeStruct((N_TEST, 16), jnp.int32),
        ],
        scratch_shapes=[pltpu.VMEM((N_TEST, N_TILES * LANES), jnp.float32)],
        compiler_params=pltpu.CompilerParams(
            dimension_semantics=("arbitrary", "arbitrary"),
        ),
    )(test_in, train_in, colvec)
    return keys3d, sel


def _sc_body(keys_hbm, sel_hbm, out_hbm, sel_v, rows_v, res_v, sem):
    w = lax.axis_index("s") * 2 + lax.axis_index("c")
    base = w * RPW
    pltpu.sync_copy(sel_hbm.at[pl.ds(base, RPW)], sel_v)
    iota16 = lax.broadcasted_iota(jnp.int32, (16,), 0)

    def row_fn(k, carry):
        r0, r1 = carry
        idxv = sel_v[k, :]
        pltpu.async_copy(keys_hbm.at[idxv], rows_v, sem).wait()
        a1 = jnp.full((16,), BIG, jnp.float32)
        a2 = jnp.full((16,), BIG, jnp.float32)
        a3 = jnp.full((16,), BIG, jnp.float32)
        for blk in range(3):
            for j in range(8):
                x = rows_v[blk, pl.ds(j * 16, 16)]
                lo = jnp.minimum(a1, x)
                hi = jnp.maximum(a1, x)
                a1 = lo
                lo = jnp.minimum(a2, hi)
                hi = jnp.maximum(a2, hi)
                a2 = lo
                a3 = jnp.minimum(a3, hi)

        def bfly_min(v):
            for s in (1, 2, 4, 8):
                perm = iota16 ^ s
                v = jnp.minimum(v, v.at[perm].get(mode="promise_in_bounds"))
            return v

        m1 = bfly_min(a1)
        a1b = jnp.where(a1 == m1, BIG, a1)
        m2 = bfly_min(jnp.minimum(a1b, a2))
        a1c = jnp.where(a1b == m2, BIG, a1b)
        a2c = jnp.where(a2 == m2, BIG, a2)
        m3 = bfly_min(jnp.minimum(a1c, jnp.minimum(a2c, a3)))

        def decode(key_f):
            ki = key_f.astype(jnp.int32)
            dist = ki >> 15
            idx = ki & 32767
            lab = (idx * 8389) >> 24
            return jnp.where(dist < MAX_DISTANCE, lab, 0)

        l1, l2, l3 = decode(m1), decode(m2), decode(m3)
        maj = jnp.where(
            l1 == l2, l1,
            jnp.where(l1 == l3, l1,
                      jnp.where(l2 == l3, l2,
                                jnp.minimum(l1, jnp.minimum(l2, l3)))))
        lane = k & 15
        sel0 = jnp.where(k < 16, lane, 99)
        sel1 = jnp.where(k < 16, 99, lane)
        r0 = jnp.where(iota16 == sel0, maj, r0)
        r1 = jnp.where(iota16 == sel1, maj, r1)
        return (r0, r1)

    r0, r1 = lax.fori_loop(
        0, RPW, row_fn,
        (jnp.zeros((16,), jnp.int32), jnp.zeros((16,), jnp.int32)))
    res_v[pl.ds(0, 16)] = r0
    res_v[pl.ds(16, 16)] = r1
    pltpu.sync_copy(res_v, out_hbm.at[pl.ds(base, RPW)])


@functools.lru_cache(maxsize=None)
def _sc_select():
    return pl.kernel(
        _sc_body,
        mesh=plsc.VectorSubcoreMesh(core_axis_name="c", subcore_axis_name="s"),
        out_type=jax.ShapeDtypeStruct((N_TEST,), jnp.int32),
        scratch_types=[
            pltpu.VMEM((RPW, 16), jnp.int32),
            pltpu.VMEM((16, LANES), jnp.float32),
            pltpu.VMEM((RPW,), jnp.int32),
            pltpu.SemaphoreType.DMA,
        ],
    )


def kernel(training_set, test_set):
    test_in = ((2 * test_set - 1) * 16384).astype(jnp.bfloat16)
    train_in = (1 - 2 * training_set).astype(jnp.bfloat16)
    train_in = jnp.pad(train_in, ((0, N_PAD - N_TRAIN), (0, 0)))
    j = jnp.arange(N_PAD, dtype=jnp.float32)
    colvec = jnp.where(j < N_TRAIN, SCALE * W + j, 2.5e7 + j).reshape(1, N_PAD)
    keys3d, sel = _tc_keys(test_in, train_in, colvec)
    keys2d = keys3d.reshape(N_BLOCKS * N_TEST, LANES)
    return _sc_select()(keys2d, sel)


# speedup vs baseline: 1.2648x; 1.2648x over previous
import functools

import jax
import jax.numpy as jnp
from jax import lax
from jax.experimental import pallas as pl
from jax.experimental.pallas import tpu as pltpu
from jax.experimental.pallas import tpu_sc as plsc

N_TRAIN = 20000
W = 256
N_TEST = 1024
BC = 2048
N_PAD = 20480
N_TILES = N_PAD // BC
BT = 512
LANES = 128
CHUNKS = BC // LANES
N_BLOCKS = N_PAD // LANES
CLASS_SIZE = 2000
MAX_DISTANCE = 256
SCALE = 16384.0
BIG = 3.0e7

N_WORKERS = 32
RPW = N_TEST // N_WORKERS


def _tc_body(test_ref, train_ref, colv_ref, keys_ref, bmin_ref):
    dot = jax.lax.dot_general(
        test_ref[...], train_ref[...],
        (((1,), (1,)), ((), ())),
        preferred_element_type=jnp.float32,
    )
    keys = dot + colv_ref[...]
    bms = []
    for c in range(CHUNKS):
        x = keys[:, c * LANES:(c + 1) * LANES]
        keys_ref[c, :, :] = x
        bms.append(jnp.min(x, axis=1, keepdims=True))
    bmin_ref[0] = jnp.concatenate(bms, axis=1)


@jax.jit
def _tc_keys(test_in, train_in, colvec):
    keys3d, bmin = pl.pallas_call(
        _tc_body,
        grid=(N_TILES, N_TEST // BT),
        in_specs=[
            pl.BlockSpec((BT, W), lambda t, i: (i, 0)),
            pl.BlockSpec((BC, W), lambda t, i: (t, 0)),
            pl.BlockSpec((1, BC), lambda t, i: (0, t)),
        ],
        out_specs=[
            pl.BlockSpec((CHUNKS, BT, LANES), lambda t, i: (t, i, 0)),
            pl.BlockSpec((1, BT, CHUNKS), lambda t, i: (t, i, 0)),
        ],
        out_shape=[
            jax.ShapeDtypeStruct((N_BLOCKS, N_TEST, LANES), jnp.float32),
            jax.ShapeDtypeStruct((N_TILES, N_TEST, CHUNKS), jnp.float32),
        ],
        compiler_params=pltpu.CompilerParams(
            dimension_semantics=("arbitrary", "arbitrary"),
        ),
    )(test_in, train_in, colvec)
    return keys3d, bmin


def _sc_body(keys_hbm, bmin_hbm, out_hbm,
             bmin_v, ids_v, rows_v, res_v, sem0, sem1):
    w = lax.axis_index("s") * 2 + lax.axis_index("c")
    base = w * RPW
    pltpu.sync_copy(bmin_hbm.at[:, pl.ds(base, RPW), :], bmin_v)
    iota16 = lax.broadcasted_iota(jnp.int32, (16,), 0)

    def bfly_min(v):
        for s in (1, 2, 4, 8):
            perm = iota16 ^ s
            v = jnp.minimum(v, v.at[perm].get(mode="promise_in_bounds"))
        return v

    def top3(slices):
        a1 = jnp.full((16,), BIG, jnp.float32)
        a2 = jnp.full((16,), BIG, jnp.float32)
        a3 = jnp.full((16,), BIG, jnp.float32)
        for x in slices:
            lo = jnp.minimum(a1, x)
            hi = jnp.maximum(a1, x)
            a1 = lo
            lo = jnp.minimum(a2, hi)
            hi = jnp.maximum(a2, hi)
            a2 = lo
            a3 = jnp.minimum(a3, hi)
        m1 = bfly_min(a1)
        a1b = jnp.where(a1 == m1, BIG, a1)
        m2 = bfly_min(jnp.minimum(a1b, a2))
        a1c = jnp.where(a1b == m2, BIG, a1b)
        a2c = jnp.where(a2 == m2, BIG, a2)
        m3 = bfly_min(jnp.minimum(a1c, jnp.minimum(a2c, a3)))
        return m1, m2, m3

    def ids_fn(k, carry):
        m1, m2, m3 = top3([bmin_v[s, k, pl.ds(0, 16)] for s in range(N_TILES)])

        def blockrow(key_f):
            ki = key_f.astype(jnp.int32)
            block = (ki & 32767) >> 7
            return block * N_TEST + (base + k)

        id1, id2, id3 = blockrow(m1), blockrow(m2), blockrow(m3)
        ids_v[k, :] = jnp.where(iota16 == 1, id2,
                                jnp.where(iota16 == 2, id3, id1))
        return carry

    lax.fori_loop(0, RPW, ids_fn, 0)

    def issue(k):
        idxv = ids_v[k, :]

        @pl.when((k & 1) == 0)
        def _():
            pltpu.async_copy(keys_hbm.at[idxv], rows_v.at[pl.ds(0, 16)], sem0)

        @pl.when((k & 1) == 1)
        def _():
            pltpu.async_copy(keys_hbm.at[idxv], rows_v.at[pl.ds(16, 16)], sem1)

    issue(jnp.int32(0))

    def row_fn(k, carry):
        r0, r1 = carry

        @pl.when(k + 1 < RPW)
        def _():
            issue(k + 1)

        par = k & 1

        @pl.when(par == 0)
        def _():
            pltpu.make_async_copy(
                keys_hbm.at[pl.ds(0, 16)], rows_v.at[pl.ds(0, 16)], sem0).wait()

        @pl.when(par == 1)
        def _():
            pltpu.make_async_copy(
                keys_hbm.at[pl.ds(0, 16)], rows_v.at[pl.ds(16, 16)], sem1).wait()

        boff = par * 16
        m1, m2, m3 = top3([rows_v[boff + blk, pl.ds(j * 16, 16)]
                           for blk in range(3) for j in range(8)])

        def decode(key_f):
            ki = key_f.astype(jnp.int32)
            dist = ki >> 15
            idx = ki & 32767
            lab = (idx * 8389) >> 24
            return jnp.where(dist < MAX_DISTANCE, lab, 0)

        l1, l2, l3 = decode(m1), decode(m2), decode(m3)
        maj = jnp.where(
            l1 == l2, l1,
            jnp.where(l1 == l3, l1,
                      jnp.where(l2 == l3, l2,
                                jnp.minimum(l1, jnp.minimum(l2, l3)))))
        lane = k & 15
        sel0 = jnp.where(k < 16, lane, 99)
        sel1 = jnp.where(k < 16, 99, lane)
        r0 = jnp.where(iota16 == sel0, maj, r0)
        r1 = jnp.where(iota16 == sel1, maj, r1)
        return (r0, r1)

    r0, r1 = lax.fori_loop(
        0, RPW, row_fn,
        (jnp.zeros((16,), jnp.int32), jnp.zeros((16,), jnp.int32)))
    res_v[pl.ds(0, 16)] = r0
    res_v[pl.ds(16, 16)] = r1
    pltpu.sync_copy(res_v, out_hbm.at[pl.ds(base, RPW)])


@functools.lru_cache(maxsize=None)
def _sc_select():
    return pl.kernel(
        _sc_body,
        mesh=plsc.VectorSubcoreMesh(core_axis_name="c", subcore_axis_name="s"),
        out_type=jax.ShapeDtypeStruct((N_TEST,), jnp.int32),
        scratch_types=[
            pltpu.VMEM((N_TILES, RPW, CHUNKS), jnp.float32),
            pltpu.VMEM((RPW, 16), jnp.int32),
            pltpu.VMEM((32, LANES), jnp.float32),
            pltpu.VMEM((RPW,), jnp.int32),
            pltpu.SemaphoreType.DMA,
            pltpu.SemaphoreType.DMA,
        ],
    )


def kernel(training_set, test_set):
    test_in = ((2 * test_set - 1) * 16384).astype(jnp.bfloat16)
    train_in = (1 - 2 * training_set).astype(jnp.bfloat16)
    train_in = jnp.pad(train_in, ((0, N_PAD - N_TRAIN), (0, 0)))
    j = jnp.arange(N_PAD, dtype=jnp.float32)
    colvec = jnp.where(j < N_TRAIN, SCALE * W + j, 2.5e7 + j).reshape(1, N_PAD)
    keys3d, bmin = _tc_keys(test_in, train_in, colvec)
    keys2d = keys3d.reshape(N_BLOCKS * N_TEST, LANES)
    return _sc_select()(keys2d, bmin)


# speedup vs baseline: 1.4930x; 1.1804x over previous
import functools

import jax
import jax.numpy as jnp
from jax import lax
from jax.experimental import pallas as pl
from jax.experimental.pallas import tpu as pltpu
from jax.experimental.pallas import tpu_sc as plsc

N_TRAIN = 20000
W = 256
N_TEST = 1024
BC = 2048
N_PAD = 20480
N_TILES = N_PAD // BC
BT = 512
LANES = 128
CHUNKS = BC // LANES
N_BLOCKS = N_PAD // LANES
CLASS_SIZE = 2000
MAX_DISTANCE = 256
SCALE = 16384.0
BIG = 3.0e7

N_WORKERS = 32
RPW = N_TEST // N_WORKERS


def _tc_body(test_ref, train_ref, colv_ref, keys_ref, bmin_ref):
    dot = jax.lax.dot_general(
        test_ref[...], train_ref[...],
        (((1,), (1,)), ((), ())),
        preferred_element_type=jnp.float32,
    )
    keys = dot + colv_ref[...]
    bms = []
    for c in range(CHUNKS):
        x = keys[:, c * LANES:(c + 1) * LANES]
        keys_ref[c, :, :] = x
        bms.append(jnp.min(x, axis=1, keepdims=True))
    bmin_ref[0] = jnp.concatenate(bms, axis=1)


@jax.jit
def _tc_keys(test_in, train_in, colvec):
    keys3d, bmin = pl.pallas_call(
        _tc_body,
        grid=(N_TILES, N_TEST // BT),
        in_specs=[
            pl.BlockSpec((BT, W), lambda t, i: (i, 0)),
            pl.BlockSpec((BC, W), lambda t, i: (t, 0)),
            pl.BlockSpec((1, BC), lambda t, i: (0, t)),
        ],
        out_specs=[
            pl.BlockSpec((CHUNKS, BT, LANES), lambda t, i: (t, i, 0)),
            pl.BlockSpec((1, BT, CHUNKS), lambda t, i: (t, i, 0)),
        ],
        out_shape=[
            jax.ShapeDtypeStruct((N_BLOCKS, N_TEST, LANES), jnp.float32),
            jax.ShapeDtypeStruct((N_TILES, N_TEST, CHUNKS), jnp.float32),
        ],
        compiler_params=pltpu.CompilerParams(
            dimension_semantics=("arbitrary", "arbitrary"),
        ),
    )(test_in, train_in, colvec)
    return keys3d, bmin


def _sc_body(keys_hbm, bmin_hbm, out_hbm,
             bmin_v, ids_v, rows_v, res_v, sem0):
    w = lax.axis_index("s") * 2 + lax.axis_index("c")
    base = w * RPW
    pltpu.sync_copy(bmin_hbm.at[:, pl.ds(base, RPW), :], bmin_v)
    iota16 = lax.broadcasted_iota(jnp.int32, (16,), 0)

    def bfly_min(v):
        for s in (1, 2, 4, 8):
            perm = iota16 ^ s
            v = jnp.minimum(v, v.at[perm].get(mode="promise_in_bounds"))
        return v

    def top3(slices):
        a1 = jnp.full((16,), BIG, jnp.float32)
        a2 = jnp.full((16,), BIG, jnp.float32)
        a3 = jnp.full((16,), BIG, jnp.float32)
        for x in slices:
            lo = jnp.minimum(a1, x)
            hi = jnp.maximum(a1, x)
            a1 = lo
            lo = jnp.minimum(a2, hi)
            hi = jnp.maximum(a2, hi)
            a2 = lo
            a3 = jnp.minimum(a3, hi)
        m1 = bfly_min(a1)
        a1b = jnp.where(a1 == m1, BIG, a1)
        m2 = bfly_min(jnp.minimum(a1b, a2))
        a1c = jnp.where(a1b == m2, BIG, a1b)
        a2c = jnp.where(a2 == m2, BIG, a2)
        m3 = bfly_min(jnp.minimum(a1c, jnp.minimum(a2c, a3)))
        return m1, m2, m3

    def ids_fn(k, carry):
        m1, m2, m3 = top3([bmin_v[s, k, pl.ds(0, 16)] for s in range(N_TILES)])

        def blockrow(key_f):
            ki = key_f.astype(jnp.int32)
            block = (ki & 32767) >> 7
            return block * N_TEST + (base + k)

        id1, id2, id3 = blockrow(m1), blockrow(m2), blockrow(m3)
        idxv = jnp.where(iota16 == 1, id2,
                         jnp.where(iota16 == 2, id3, id1))
        ids_v[k, :] = idxv
        pltpu.async_copy(keys_hbm.at[idxv], rows_v.at[pl.ds(k * 16, 16)], sem0)
        return carry

    lax.fori_loop(0, RPW, ids_fn, 0)
    pltpu.make_async_copy(
        keys_hbm.at[pl.ds(0, RPW * 16)], rows_v, sem0).wait()

    def row_fn(k, carry):
        r0, r1 = carry
        m1, m2, m3 = top3([rows_v[k * 16 + blk, pl.ds(j * 16, 16)]
                           for blk in range(3) for j in range(8)])

        def decode(key_f):
            ki = key_f.astype(jnp.int32)
            dist = ki >> 15
            idx = ki & 32767
            lab = (idx * 8389) >> 24
            return jnp.where(dist < MAX_DISTANCE, lab, 0)

        l1, l2, l3 = decode(m1), decode(m2), decode(m3)
        maj = jnp.where(
            l1 == l2, l1,
            jnp.where(l1 == l3, l1,
                      jnp.where(l2 == l3, l2,
                                jnp.minimum(l1, jnp.minimum(l2, l3)))))
        lane = k & 15
        sel0 = jnp.where(k < 16, lane, 99)
        sel1 = jnp.where(k < 16, 99, lane)
        r0 = jnp.where(iota16 == sel0, maj, r0)
        r1 = jnp.where(iota16 == sel1, maj, r1)
        return (r0, r1)

    r0, r1 = lax.fori_loop(
        0, RPW, row_fn,
        (jnp.zeros((16,), jnp.int32), jnp.zeros((16,), jnp.int32)))
    res_v[pl.ds(0, 16)] = r0
    res_v[pl.ds(16, 16)] = r1
    pltpu.sync_copy(res_v, out_hbm.at[pl.ds(base, RPW)])


@functools.lru_cache(maxsize=None)
def _sc_select():
    return pl.kernel(
        _sc_body,
        mesh=plsc.VectorSubcoreMesh(core_axis_name="c", subcore_axis_name="s"),
        out_type=jax.ShapeDtypeStruct((N_TEST,), jnp.int32),
        scratch_types=[
            pltpu.VMEM((N_TILES, RPW, CHUNKS), jnp.float32),
            pltpu.VMEM((RPW, 16), jnp.int32),
            pltpu.VMEM((RPW * 16, LANES), jnp.float32),
            pltpu.VMEM((RPW,), jnp.int32),
            pltpu.SemaphoreType.DMA,
        ],
    )


def kernel(training_set, test_set):
    test_in = ((2 * test_set - 1) * 16384).astype(jnp.bfloat16)
    train_in = (1 - 2 * training_set).astype(jnp.bfloat16)
    train_in = jnp.pad(train_in, ((0, N_PAD - N_TRAIN), (0, 0)))
    j = jnp.arange(N_PAD, dtype=jnp.float32)
    colvec = jnp.where(j < N_TRAIN, SCALE * W + j, 2.5e7 + j).reshape(1, N_PAD)
    keys3d, bmin = _tc_keys(test_in, train_in, colvec)
    keys2d = keys3d.reshape(N_BLOCKS * N_TEST, LANES)
    return _sc_select()(keys2d, bmin)
